# baseline (speedup 1.0000x reference)
import functools

import jax
import jax.numpy as jnp
from jax import lax
from jax.experimental import pallas as pl
from jax.experimental.pallas import tpu as pltpu
from jax.experimental.pallas import tpu_sc as plsc

N = 50000
E = 800000
H = 4
C = 16
HID = 64
LAT = 32
DH = 256

NBLK = 1000
EBLK = 1000
CH = 128
NCH = E // CH


def _leaky(v):
    return jnp.where(v >= 0, v, 0.2 * v)


@functools.lru_cache(maxsize=None)
def _sc_info():
    info = plsc.get_sparse_core_info()
    return info.num_cores, info.num_subcores


def _sc_mesh():
    return plsc.VectorSubcoreMesh(core_axis_name="c", subcore_axis_name="s")


_SC_PARAMS = pltpu.CompilerParams(use_tc_tiling_on_sc=False)


def _edge_body(ea_ref, w_ref, b_ref, v3_ref, ad_ref, a8_ref):
    eh = jax.nn.relu(ea_ref[...] @ w_ref[...] + b_ref[...])
    a12 = eh @ v3_ref[...]
    one = jnp.ones((EBLK, 1), jnp.float32)
    zpad = jnp.zeros((EBLK, 3), jnp.float32)
    ad_ref[...] = jnp.concatenate([a12, one, zpad], axis=1)
    a8_ref[...] = jnp.stack(
        [a12[:, 4 * l:4 * l + 4] for l in range(3)], axis=0)


def _edge_dense(edge_attr, ee_W, ee_b, V3):
    return pl.pallas_call(
        _edge_body,
        grid=(E // EBLK,),
        in_specs=[
            pl.BlockSpec((EBLK, 2), lambda i: (i, 0)),
            pl.BlockSpec((2, HID), lambda i: (0, 0)),
            pl.BlockSpec((1, HID), lambda i: (0, 0)),
            pl.BlockSpec((HID, 12), lambda i: (0, 0)),
        ],
        out_specs=[
            pl.BlockSpec((EBLK, 16), lambda i: (i, 0)),
            pl.BlockSpec((3, EBLK, 4), lambda i: (0, i, 0)),
        ],
        out_shape=[
            jax.ShapeDtypeStruct((E, 16), jnp.float32),
            jax.ShapeDtypeStruct((3, E, 4), jnp.float32),
        ],
    )(edge_attr, ee_W, ee_b.reshape(1, HID), V3)


def _prep_block(h, W, us, ud):
    xp = h @ W
    z12 = jnp.zeros((h.shape[0], 12), jnp.float32)
    a_s = jnp.concatenate([h @ us, z12], axis=1)
    a_d = jnp.concatenate([h @ ud, z12], axis=1)
    return xp[:, :32], xp[:, 32:], a_s, a_d


def _encprep_body(x_ref, ft_ref, emb_ref, w_ref, b_ref, gw_ref, us_ref,
                  ud_ref, xp0_ref, xp1_ref, as_ref, ad_ref):
    ft = ft_ref[...]
    oh = (ft == lax.broadcasted_iota(jnp.int32, (ft.shape[0], 3), 1))
    fe = oh.astype(jnp.float32) @ emb_ref[...]
    hin = jnp.concatenate([x_ref[...], fe], axis=-1)
    h = jax.nn.relu(hin @ w_ref[...] + b_ref[...])
    xp0_ref[...], xp1_ref[...], as_ref[...], ad_ref[...] = _prep_block(
        h, gw_ref[...], us_ref[...], ud_ref[...])


def _encprep(x, face_types, emb, ne_W, ne_b, gW, us, ud):
    return pl.pallas_call(
        _encprep_body,
        grid=(N // NBLK,),
        in_specs=[
            pl.BlockSpec((NBLK, 9), lambda i: (i, 0)),
            pl.BlockSpec((NBLK, 1), lambda i: (i, 0)),
            pl.BlockSpec((3, 8), lambda i: (0, 0)),
            pl.BlockSpec((17, HID), lambda i: (0, 0)),
            pl.BlockSpec((1, HID), lambda i: (0, 0)),
            pl.BlockSpec((HID, HID), lambda i: (0, 0)),
            pl.BlockSpec((HID, H), lambda i: (0, 0)),
            pl.BlockSpec((HID, H), lambda i: (0, 0)),
        ],
        out_specs=[
            pl.BlockSpec((NBLK, 32), lambda i: (i, 0)),
            pl.BlockSpec((NBLK, 32), lambda i: (i, 0)),
            pl.BlockSpec((NBLK, 16), lambda i: (i, 0)),
            pl.BlockSpec((NBLK, 16), lambda i: (i, 0)),
        ],
        out_shape=[
            jax.ShapeDtypeStruct((N, 32), jnp.float32),
            jax.ShapeDtypeStruct((N, 32), jnp.float32),
            jax.ShapeDtypeStruct((N, 16), jnp.float32),
            jax.ShapeDtypeStruct((N, 16), jnp.float32),
        ],
    )(x, face_types.astype(jnp.int32).reshape(N, 1), emb, ne_W,
      ne_b.reshape(1, HID), gW, us, ud)


def _fin_block(l, acc36, acc32, st, a_s, a_d, xp0, xp1, b):
    a36 = acc36[0] + acc36[1]
    a32 = acc32[0] + acc32[1]
    stt = st[0] + st[1]
    deg = jnp.maximum(stt[:, 12:13], 1.0)
    aloop = stt[:, 4 * l:4 * l + 4] / deg
    wself = jnp.exp(_leaky(a_s[:, :4] + a_d[:, :4] + aloop))
    s = a36[:, 0:4] + wself
    acc = jnp.concatenate([a36[:, 4:36], a32], axis=1)
    B = acc.shape[0]
    xp = jnp.concatenate([xp0, xp1], axis=1).reshape(B, H, C)
    accf = acc.reshape(B, H, C) + xp * wself[:, :, None]
    out = accf / (s + 1e-16)[:, :, None]
    return jax.nn.relu(out.reshape(B, HID) + b)


def _make_finprep(l):
    def body(acc36_ref, acc32_ref, st_ref, as_ref, ad_ref, xp0_ref,
             xp1_ref, b_ref, gw_ref, us_ref, ud_ref,
             oxp0_ref, oxp1_ref, oas_ref, oad_ref):
        h = _fin_block(l, acc36_ref[...], acc32_ref[...], st_ref[...],
                       as_ref[...], ad_ref[...], xp0_ref[...],
                       xp1_ref[...], b_ref[...])
        oxp0_ref[...], oxp1_ref[...], oas_ref[...], oad_ref[...] = (
            _prep_block(h, gw_ref[...], us_ref[...], ud_ref[...]))
    return body


def _finprep(l, acc36, acc32, stats, a_s, a_d, xp0, xp1, b, gW, us, ud):
    return pl.pallas_call(
        _make_finprep(l),
        grid=(N // NBLK,),
        in_specs=[
            pl.BlockSpec((2, NBLK, 36), lambda i: (0, i, 0)),
            pl.BlockSpec((2, NBLK, 32), lambda i: (0, i, 0)),
            pl.BlockSpec((2, NBLK, 16), lambda i: (0, i, 0)),
            pl.BlockSpec((NBLK, 16), lambda i: (i, 0)),
            pl.BlockSpec((NBLK, 16), lambda i: (i, 0)),
            pl.BlockSpec((NBLK, 32), lambda i: (i, 0)),
            pl.BlockSpec((NBLK, 32), lambda i: (i, 0)),
            pl.BlockSpec((1, HID), lambda i: (0, 0)),
            pl.BlockSpec((HID, HID), lambda i: (0, 0)),
            pl.BlockSpec((HID, H), lambda i: (0, 0)),
            pl.BlockSpec((HID, H), lambda i: (0, 0)),
        ],
        out_specs=[
            pl.BlockSpec((NBLK, 32), lambda i: (i, 0)),
            pl.BlockSpec((NBLK, 32), lambda i: (i, 0)),
            pl.BlockSpec((NBLK, 16), lambda i: (i, 0)),
            pl.BlockSpec((NBLK, 16), lambda i: (i, 0)),
        ],
        out_shape=[
            jax.ShapeDtypeStruct((N, 32), jnp.float32),
            jax.ShapeDtypeStruct((N, 32), jnp.float32),
            jax.ShapeDtypeStruct((N, 16), jnp.float32),
            jax.ShapeDtypeStruct((N, 16), jnp.float32),
        ],
    )(acc36, acc32, stats, a_s, a_d, xp0, xp1, b.reshape(1, HID), gW,
      us, ud)


def _make_fin2g(l):
    def body(acc36_ref, acc32_ref, st_ref, as_ref, ad_ref, xp0_ref,
             xp1_ref, b_ref, g_ref):
        h = _fin_block(l, acc36_ref[...], acc32_ref[...], st_ref[...],
                       as_ref[...], ad_ref[...], xp0_ref[...],
                       xp1_ref[...], b_ref[...])

        @pl.when(pl.program_id(0) == 0)
        def _():
            g_ref[...] = jnp.zeros_like(g_ref)

        g_ref[...] += jnp.sum(h, axis=0, keepdims=True)
    return body


def _fin2g(l, acc36, acc32, stats, a_s, a_d, xp0, xp1, b):
    return pl.pallas_call(
        _make_fin2g(l),
        grid=(N // NBLK,),
        in_specs=[
            pl.BlockSpec((2, NBLK, 36), lambda i: (0, i, 0)),
            pl.BlockSpec((2, NBLK, 32), lambda i: (0, i, 0)),
            pl.BlockSpec((2, NBLK, 16), lambda i: (0, i, 0)),
            pl.BlockSpec((NBLK, 16), lambda i: (i, 0)),
            pl.BlockSpec((NBLK, 16), lambda i: (i, 0)),
            pl.BlockSpec((NBLK, 32), lambda i: (i, 0)),
            pl.BlockSpec((NBLK, 32), lambda i: (i, 0)),
            pl.BlockSpec((1, HID), lambda i: (0, 0)),
        ],
        out_specs=pl.BlockSpec((1, HID), lambda i: (0, 0)),
        out_shape=jax.ShapeDtypeStruct((1, HID), jnp.float32),
    )(acc36, acc32, stats, a_s, a_d, xp0, xp1, b.reshape(1, HID))


_SIGCOLS = (5, 10, 11, 16, 17)


def _dec_body(g_ref, muW_ref, mub_ref, lvW_ref, lvb_ref, d0W_ref, d0b_ref,
              d0g_ref, d0e_ref, d1W_ref, d1b_ref, d1g_ref, d1e_ref,
              d2W_ref, d2b_ref, d2g_ref, d2e_ref, hW_ref, hb_ref,
              aW_ref, ab_ref, ho_ref, aux_ref, mu_ref, lv_ref):
    g = g_ref[...] / N
    mu = g @ muW_ref[...] + mub_ref[...]
    lv = g @ lvW_ref[...] + lvb_ref[...]
    hd = mu
    for dW, db, dg, de in ((d0W_ref, d0b_ref, d0g_ref, d0e_ref),
                           (d1W_ref, d1b_ref, d1g_ref, d1e_ref),
                           (d2W_ref, d2b_ref, d2g_ref, d2e_ref)):
        hd = hd @ dW[...] + db[...]
        mn = hd.mean(-1, keepdims=True)
        vr = ((hd - mn) ** 2).mean(-1, keepdims=True)
        hd = (hd - mn) / jnp.sqrt(vr + 1e-5) * dg[...] + de[...]
        hd = jax.nn.relu(hd)
    ho = hd @ hW_ref[...] + hb_ref[...]
    col = lax.broadcasted_iota(jnp.int32, ho.shape, 1)
    sigmask = jnp.zeros(ho.shape, jnp.bool_)
    for c in _SIGCOLS:
        sigmask = jnp.logical_or(sigmask, col == c)
    ho_ref[...] = jnp.where(sigmask, jax.nn.sigmoid(ho), ho)
    aux_ref[...] = mu @ aW_ref[...] + ab_ref[...]
    mu_ref[...] = mu
    lv_ref[...] = lv


def _decode(gsum, p, headsW, headsb):
    full = lambda s: pl.BlockSpec(s, lambda: tuple(0 for _ in s))
    args = (gsum, p["mu_W"], p["mu_b"].reshape(1, LAT), p["lv_W"],
            p["lv_b"].reshape(1, LAT),
            p["d0_W"], p["d0_b"].reshape(1, DH), p["d0_g"].reshape(1, DH),
            p["d0_be"].reshape(1, DH),
            p["d1_W"], p["d1_b"].reshape(1, DH), p["d1_g"].reshape(1, DH),
            p["d1_be"].reshape(1, DH),
            p["d2_W"], p["d2_b"].reshape(1, DH), p["d2_g"].reshape(1, DH),
            p["d2_be"].reshape(1, DH),
            headsW, headsb, p["aux_W"], p["aux_b"].reshape(1, 4))
    return pl.pallas_call(
        _dec_body,
        in_specs=[full(a.shape) for a in args],
        out_specs=[full((1, 18)), full((1, 4)), full((1, LAT)),
                   full((1, LAT))],
        out_shape=[
            jax.ShapeDtypeStruct((1, 18), jnp.float32),
            jax.ShapeDtypeStruct((1, 4), jnp.float32),
            jax.ShapeDtypeStruct((1, LAT), jnp.float32),
            jax.ShapeDtypeStruct((1, LAT), jnp.float32),
        ],
    )(*args)


def _sc_stats_build(NC, NS):
    NW = NC * NS
    MAXJ = -(-NCH // NW)

    @functools.partial(
        pl.kernel,
        out_type=jax.ShapeDtypeStruct((NC, N, 16), jnp.float32),
        mesh=_sc_mesh(),
        compiler_params=_SC_PARAMS,
        scratch_types=dict(
            idxd=pltpu.VMEM((1, CH), jnp.int32),
            rows=pltpu.VMEM((CH, 16), jnp.float32),
            statS=pltpu.VMEM_SHARED((N, 16), jnp.float32),
        ),
    )
    def k(ad_h, dst2_h, z16_h, out_h, *, idxd, rows, statS):
        c = lax.axis_index("c")
        s = lax.axis_index("s")
        wid = s * NC + c

        @pl.when(s == 0)
        def _():
            pltpu.sync_copy(z16_h, statS)

        plsc.subcore_barrier()

        def chunk(j, _):
            cid = j * NW + wid

            @pl.when(cid < NCH)
            def _():
                pltpu.sync_copy(dst2_h.at[pl.ds(cid, 1)], idxd)
                pltpu.sync_copy(ad_h.at[pl.ds(cid * CH, CH)], rows)
                pltpu.sync_copy(rows, statS.at[idxd.at[0]], add=True)

            return ()

        lax.fori_loop(0, MAXJ, chunk, ())
        plsc.subcore_barrier()

        @pl.when(s == 0)
        def _():
            pltpu.sync_copy(statS, out_h.at[c])

    return k


CH0 = 64
EP0 = 802816
NCH0P = EP0 // CH0


def _sc_pass0_build(NC, NS):
    NW = NC * NS
    MAXJP = NCH0P // NW // 2

    @functools.partial(
        pl.kernel,
        out_type=[
            jax.ShapeDtypeStruct((EP0, 16), jnp.float32),
            jax.ShapeDtypeStruct((NC, N, 36), jnp.float32),
        ],
        mesh=_sc_mesh(),
        compiler_params=_SC_PARAMS,
        scratch_types=dict(
            idxsA=pltpu.VMEM((1, CH0), jnp.int32),
            idxdA=pltpu.VMEM((1, CH0), jnp.int32),
            aehgA=pltpu.VMEM((CH0 * 4 + 16,), jnp.float32),
            asrcgA=pltpu.VMEM((CH0, 16), jnp.float32),
            adstgA=pltpu.VMEM((CH0, 16), jnp.float32),
            xpgA=pltpu.VMEM((CH0, 32), jnp.float32),
            srowA=pltpu.VMEM((CH0, 36), jnp.float32),
            wrowA=pltpu.VMEM((CH0, 16), jnp.float32),
            idxsB=pltpu.VMEM((1, CH0), jnp.int32),
            idxdB=pltpu.VMEM((1, CH0), jnp.int32),
            aehgB=pltpu.VMEM((CH0 * 4 + 16,), jnp.float32),
            asrcgB=pltpu.VMEM((CH0, 16), jnp.float32),
            adstgB=pltpu.VMEM((CH0, 16), jnp.float32),
            xpgB=pltpu.VMEM((CH0, 32), jnp.float32),
            srowB=pltpu.VMEM((CH0, 36), jnp.float32),
            wrowB=pltpu.VMEM((CH0, 16), jnp.float32),
            accS=pltpu.VMEM_SHARED((N, 36), jnp.float32),
            gsemA=pltpu.SemaphoreType.DMA,
            gsemA2=pltpu.SemaphoreType.DMA,
            gsemA3=pltpu.SemaphoreType.DMA,
            gsemB=pltpu.SemaphoreType.DMA,
            gsemB2=pltpu.SemaphoreType.DMA,
            gsemB3=pltpu.SemaphoreType.DMA,
            ssemA=pltpu.SemaphoreType.DMA,
            ssemB=pltpu.SemaphoreType.DMA,
            wsemA=pltpu.SemaphoreType.DMA,
            wsemB=pltpu.SemaphoreType.DMA,
        ),
    )
    def k(src2_h, dst2_h, aeh4_h, asrc_h, adst_h, xp0_h, z36_h,
          w_out, acc_out, *, idxsA, idxdA, aehgA, asrcgA, adstgA, xpgA,
          srowA, wrowA, idxsB, idxdB, aehgB, asrcgB, adstgB, xpgB,
          srowB, wrowB, accS, gsemA, gsemA2, gsemA3, gsemB, gsemB2,
          gsemB3, ssemA, ssemB, wsemA, wsemB):
        c = lax.axis_index("c")
        s = lax.axis_index("s")
        wid = s * NC + c

        @pl.when(s == 0)
        def _():
            pltpu.sync_copy(z36_h, accS)

        plsc.subcore_barrier()

        def load_and_gather(cid, idxs, idxd, aehg, asrcg, adstg, xpg,
                            gs1, gs2, gs3):
            pltpu.sync_copy(src2_h.at[pl.ds(cid, 1)], idxs)
            pltpu.sync_copy(dst2_h.at[pl.ds(cid, 1)], idxd)
            pltpu.sync_copy(aeh4_h.at[pl.ds(cid * CH0 * 4, CH0 * 4)],
                            aehg.at[pl.ds(0, CH0 * 4)])
            g1 = pltpu.async_copy(asrc_h.at[idxs.at[0]], asrcg, gs1)
            g2 = pltpu.async_copy(adst_h.at[idxd.at[0]], adstg, gs2)
            g3 = pltpu.async_copy(xp0_h.at[idxs.at[0]], xpg, gs3)
            return g1, g2, g3

        def compute(aehg, asrcg, adstg, xpg, srow, wrow):
            def estep(e, _):
                al = (asrcg[e, :] + adstg[e, :] + aehg[pl.ds(e * 4, 16)])
                al = jnp.where(al >= 0, al, 0.2 * al)
                w16 = jnp.exp(al)
                wrow[e, :] = w16
                srow[e, pl.ds(0, 16)] = w16
                x0 = xpg[e, pl.ds(0, 16)]
                x1 = xpg[e, pl.ds(16, 16)]
                srow[e, pl.ds(4, 16)] = x0 * w16[0]
                srow[e, pl.ds(20, 16)] = x1 * w16[1]
                return ()

            lax.fori_loop(0, CH0, estep, (), unroll=4)

        def pair(jj, _):
            cA = (2 * jj) * NW + wid
            cB = cA + NW
            gA = load_and_gather(cA, idxsA, idxdA, aehgA, asrcgA,
                                 adstgA, xpgA, gsemA, gsemA2, gsemA3)
            for g in gA:
                g.wait()
            compute(aehgA, asrcgA, adstgA, xpgA, srowA, wrowA)
            oA1 = pltpu.async_copy(wrowA,
                                   w_out.at[pl.ds(cA * CH0, CH0)], wsemA)
            oA2 = pltpu.async_copy(srowA, accS.at[idxdA.at[0]], ssemA,
                                   add=True)

            gB = load_and_gather(cB, idxsB, idxdB, aehgB, asrcgB,
                                 adstgB, xpgB, gsemB, gsemB2, gsemB3)
            for g in gB:
                g.wait()
            compute(aehgB, asrcgB, adstgB, xpgB, srowB, wrowB)
            oB1 = pltpu.async_copy(wrowB,
                                   w_out.at[pl.ds(cB * CH0, CH0)], wsemB)
            oB2 = pltpu.async_copy(srowB, accS.at[idxdB.at[0]], ssemB,
                                   add=True)
            oB1.wait()
            oB2.wait()
            oA1.wait()
            oA2.wait()
            return ()

        lax.fori_loop(0, MAXJP, pair, ())
        plsc.subcore_barrier()

        @pl.when(s == 0)
        def _():
            pltpu.sync_copy(accS, acc_out.at[c])

    return k


def _sc_pass1_build(NC, NS):
    NW = NC * NS
    MAXJP = (-(-NCH // NW) + 1) // 2

    @functools.partial(
        pl.kernel,
        out_type=jax.ShapeDtypeStruct((NC, N, 32), jnp.float32),
        mesh=_sc_mesh(),
        compiler_params=_SC_PARAMS,
        scratch_types=dict(
            idxsA=pltpu.VMEM((1, CH), jnp.int32),
            idxdA=pltpu.VMEM((1, CH), jnp.int32),
            wgA=pltpu.VMEM((CH, 16), jnp.float32),
            xpgA=pltpu.VMEM((CH, 32), jnp.float32),
            srowA=pltpu.VMEM((CH, 32), jnp.float32),
            idxsB=pltpu.VMEM((1, CH), jnp.int32),
            idxdB=pltpu.VMEM((1, CH), jnp.int32),
            wgB=pltpu.VMEM((CH, 16), jnp.float32),
            xpgB=pltpu.VMEM((CH, 32), jnp.float32),
            srowB=pltpu.VMEM((CH, 32), jnp.float32),
            accS=pltpu.VMEM_SHARED((N, 32), jnp.float32),
            gsemA=pltpu.SemaphoreType.DMA,
            gsemB=pltpu.SemaphoreType.DMA,
            ssemA=pltpu.SemaphoreType.DMA,
            ssemB=pltpu.SemaphoreType.DMA,
        ),
    )
    def k(src2_h, dst2_h, w_h, xp1_h, z32_h, acc_out, *, idxsA, idxdA,
          wgA, xpgA, srowA, idxsB, idxdB, wgB, xpgB, srowB, accS,
          gsemA, gsemB, ssemA, ssemB):
        c = lax.axis_index("c")
        s = lax.axis_index("s")
        wid = s * NC + c

        @pl.when(s == 0)
        def _():
            pltpu.sync_copy(z32_h, accS)

        plsc.subcore_barrier()

        def load_and_gather(cid, idxs, idxd, wg, xpg, gsem):
            pltpu.sync_copy(src2_h.at[pl.ds(cid, 1)], idxs)
            pltpu.sync_copy(dst2_h.at[pl.ds(cid, 1)], idxd)
            pltpu.sync_copy(w_h.at[pl.ds(cid * CH, CH)], wg)
            return pltpu.async_copy(xp1_h.at[idxs.at[0]], xpg, gsem)

        def compute(wg, xpg, srow):
            def estep(e, _):
                v = wg[e, :]
                srow[e, pl.ds(0, 16)] = xpg[e, pl.ds(0, 16)] * v[2]
                srow[e, pl.ds(16, 16)] = xpg[e, pl.ds(16, 16)] * v[3]
                return ()

            lax.fori_loop(0, CH, estep, (), unroll=4)

        def pair(jj, _):
            cA = (2 * jj) * NW + wid
            cB = cA + NW
            gA = load_and_gather(cA, idxsA, idxdA, wgA, xpgA, gsemA)
            bvalid = cB < NCH

            @pl.when(bvalid)
            def _():
                load_and_gather(cB, idxsB, idxdB, wgB, xpgB, gsemB)

            gA.wait()
            compute(wgA, xpgA, srowA)
            oA = pltpu.async_copy(srowA, accS.at[idxdA.at[0]], ssemA,
                                  add=True)

            @pl.when(bvalid)
            def _():
                pltpu.make_async_copy(xp1_h.at[idxsB.at[0]], xpgB,
                                      gsemB).wait()
                compute(wgB, xpgB, srowB)
                cpB = pltpu.async_copy(srowB, accS.at[idxdB.at[0]],
                                       ssemB, add=True)
                cpB.wait()

            oA.wait()
            return ()

        lax.fori_loop(0, MAXJP, pair, ())
        plsc.subcore_barrier()

        @pl.when(s == 0)
        def _():
            pltpu.sync_copy(accS, acc_out.at[c])

    return k


@functools.lru_cache(maxsize=None)
def _sc_kernels():
    NC, NS = _sc_info()
    return (_sc_stats_build(NC, NS), _sc_pass0_build(NC, NS),
            _sc_pass1_build(NC, NS))


def kernel(x, face_types, edge_index, edge_attr, params):
    p = params
    srci = edge_index[0].astype(jnp.int32)
    dsti = edge_index[1].astype(jnp.int32)
    src2 = srci.reshape(NCH, CH)
    dst2 = dsti.reshape(NCH, CH)
    padz = jnp.zeros((EP0 - E,), jnp.int32)
    src2p0 = jnp.concatenate([srci, padz]).reshape(NCH0P, CH0)
    dst2p0 = jnp.concatenate([dsti, padz]).reshape(NCH0P, CH0)

    def fold(W, a):
        return (W.reshape(HID, H, C) * a[0][None]).sum(-1)

    us = [fold(p["g%d_W" % l], p["g%d_as" % l]) for l in range(3)]
    ud = [fold(p["g%d_W" % l], p["g%d_ad" % l]) for l in range(3)]
    ve = [fold(p["g%d_We" % l], p["g%d_ae" % l]) for l in range(3)]
    V3 = jnp.concatenate(ve, axis=1)
    headsW = jnp.concatenate(
        [p[n + "_W"] for n in ("core", "fil", "file", "h10", "h11",
                               "h1e", "h20", "h21", "h2e")], axis=1)
    headsb = jnp.concatenate(
        [p[n + "_b"] for n in ("core", "fil", "file", "h10", "h11",
                               "h1e", "h20", "h21", "h2e")]).reshape(1, 18)

    k_stats, k_pass0, k_pass1 = _sc_kernels()

    z16 = jnp.zeros((N, 16), jnp.float32)
    z32 = jnp.zeros((N, 32), jnp.float32)
    z36 = jnp.zeros((N, 36), jnp.float32)

    aehdeg, aeh8 = _edge_dense(edge_attr, p["ee_W"], p["ee_b"], V3)
    stats = k_stats(aehdeg, dst2, z16)

    xp0, xp1, a_s, a_d = _encprep(x, face_types, p["emb"], p["ne_W"],
                                  p["ne_b"], p["g0_W"], us[0], ud[0])

    for l in range(3):
        aeh4f = jnp.concatenate(
            [aeh8[l], jnp.full((EP0 - E, 4), -1e30, jnp.float32)]
        ).reshape(EP0 * 4)
        wrows, acc36 = k_pass0(src2p0, dst2p0, aeh4f, a_s, a_d, xp0, z36)
        acc32 = k_pass1(src2, dst2, wrows, xp1, z32)
        b = p["g%d_b" % l]
        if l < 2:
            xp0, xp1, a_s, a_d = _finprep(
                l, acc36, acc32, stats, a_s, a_d, xp0, xp1, b,
                p["g%d_W" % (l + 1)], us[l + 1], ud[l + 1])
        else:
            gsum = _fin2g(l, acc36, acc32, stats, a_s, a_d, xp0, xp1, b)

    ho, aux, mu, lv = _decode(gsum, p, headsW, headsb)

    core = ho[:, 0:4]
    fr = ho[:, 4:5]
    fx = ho[:, 5:6]
    h1 = jnp.stack([ho[:, 6:8], ho[:, 8:10]], axis=1)
    h1e = ho[:, 10:12]
    h2 = jnp.stack([ho[:, 12:14], ho[:, 14:16]], axis=1)
    h2e = ho[:, 16:18]
    return (core, fr, fx, h1, h1e, h2, h2e, aux, mu, lv)


# speedup vs baseline: 1.1683x; 1.1683x over previous
import functools

import jax
import jax.numpy as jnp
from jax import lax
from jax.experimental import pallas as pl
from jax.experimental.pallas import tpu as pltpu
from jax.experimental.pallas import tpu_sc as plsc

N = 50000
E = 800000
H = 4
C = 16
HID = 64
LAT = 32
DH = 256

NBLK = 1000
EBLK = 1000
CH = 128
NCH = E // CH


def _leaky(v):
    return jnp.where(v >= 0, v, 0.2 * v)


@functools.lru_cache(maxsize=None)
def _sc_info():
    info = plsc.get_sparse_core_info()
    return info.num_cores, info.num_subcores


def _sc_mesh():
    return plsc.VectorSubcoreMesh(core_axis_name="c", subcore_axis_name="s")


_SC_PARAMS = pltpu.CompilerParams(use_tc_tiling_on_sc=False)


def _edge_body(ea_ref, w_ref, b_ref, v3_ref, ad_ref, a8_ref):
    eh = jax.nn.relu(ea_ref[...] @ w_ref[...] + b_ref[...])
    a12 = eh @ v3_ref[...]
    one = jnp.ones((EBLK, 1), jnp.float32)
    zpad = jnp.zeros((EBLK, 3), jnp.float32)
    ad_ref[...] = jnp.concatenate([a12, one, zpad], axis=1)
    z4 = jnp.zeros((EBLK, 4), jnp.float32)
    a8_ref[...] = jnp.stack(
        [jnp.concatenate([a12[:, 4 * l:4 * l + 4], z4], axis=1)
         for l in range(3)], axis=0)


def _edge_dense(edge_attr, ee_W, ee_b, V3):
    return pl.pallas_call(
        _edge_body,
        grid=(E // EBLK,),
        in_specs=[
            pl.BlockSpec((EBLK, 2), lambda i: (i, 0)),
            pl.BlockSpec((2, HID), lambda i: (0, 0)),
            pl.BlockSpec((1, HID), lambda i: (0, 0)),
            pl.BlockSpec((HID, 12), lambda i: (0, 0)),
        ],
        out_specs=[
            pl.BlockSpec((EBLK, 16), lambda i: (i, 0)),
            pl.BlockSpec((3, EBLK, 8), lambda i: (0, i, 0)),
        ],
        out_shape=[
            jax.ShapeDtypeStruct((E, 16), jnp.float32),
            jax.ShapeDtypeStruct((3, E, 8), jnp.float32),
        ],
    )(edge_attr, ee_W, ee_b.reshape(1, HID), V3)


def _prep_block(h, W, us, ud):
    xp = h @ W
    z12 = jnp.zeros((h.shape[0], 12), jnp.float32)
    a_s = jnp.concatenate([h @ us, z12], axis=1)
    a_d = jnp.concatenate([h @ ud, z12], axis=1)
    return xp[:, :32], xp[:, 32:], a_s, a_d


def _encprep_body(x_ref, ft_ref, emb_ref, w_ref, b_ref, gw_ref, us_ref,
                  ud_ref, xp0_ref, xp1_ref, as_ref, ad_ref):
    ft = ft_ref[...]
    oh = (ft == lax.broadcasted_iota(jnp.int32, (ft.shape[0], 3), 1))
    fe = oh.astype(jnp.float32) @ emb_ref[...]
    hin = jnp.concatenate([x_ref[...], fe], axis=-1)
    h = jax.nn.relu(hin @ w_ref[...] + b_ref[...])
    xp0_ref[...], xp1_ref[...], as_ref[...], ad_ref[...] = _prep_block(
        h, gw_ref[...], us_ref[...], ud_ref[...])


def _encprep(x, face_types, emb, ne_W, ne_b, gW, us, ud):
    return pl.pallas_call(
        _encprep_body,
        grid=(N // NBLK,),
        in_specs=[
            pl.BlockSpec((NBLK, 9), lambda i: (i, 0)),
            pl.BlockSpec((NBLK, 1), lambda i: (i, 0)),
            pl.BlockSpec((3, 8), lambda i: (0, 0)),
            pl.BlockSpec((17, HID), lambda i: (0, 0)),
            pl.BlockSpec((1, HID), lambda i: (0, 0)),
            pl.BlockSpec((HID, HID), lambda i: (0, 0)),
            pl.BlockSpec((HID, H), lambda i: (0, 0)),
            pl.BlockSpec((HID, H), lambda i: (0, 0)),
        ],
        out_specs=[
            pl.BlockSpec((NBLK, 32), lambda i: (i, 0)),
            pl.BlockSpec((NBLK, 32), lambda i: (i, 0)),
            pl.BlockSpec((NBLK, 16), lambda i: (i, 0)),
            pl.BlockSpec((NBLK, 16), lambda i: (i, 0)),
        ],
        out_shape=[
            jax.ShapeDtypeStruct((N, 32), jnp.float32),
            jax.ShapeDtypeStruct((N, 32), jnp.float32),
            jax.ShapeDtypeStruct((N, 16), jnp.float32),
            jax.ShapeDtypeStruct((N, 16), jnp.float32),
        ],
    )(x, face_types.astype(jnp.int32).reshape(N, 1), emb, ne_W,
      ne_b.reshape(1, HID), gW, us, ud)


def _fin_block(l, acc36, acc32, st, a_s, a_d, xp0, xp1, b):
    a36 = acc36[0] + acc36[1]
    a32 = acc32[0] + acc32[1]
    stt = st[0] + st[1]
    deg = jnp.maximum(stt[:, 12:13], 1.0)
    aloop = stt[:, 4 * l:4 * l + 4] / deg
    wself = jnp.exp(_leaky(a_s[:, :4] + a_d[:, :4] + aloop))
    s = a36[:, 0:4] + wself
    acc = jnp.concatenate([a36[:, 4:36], a32], axis=1)
    B = acc.shape[0]
    xp = jnp.concatenate([xp0, xp1], axis=1).reshape(B, H, C)
    accf = acc.reshape(B, H, C) + xp * wself[:, :, None]
    out = accf / (s + 1e-16)[:, :, None]
    return jax.nn.relu(out.reshape(B, HID) + b)


def _make_finprep(l):
    def body(acc36_ref, acc32_ref, st_ref, as_ref, ad_ref, xp0_ref,
             xp1_ref, b_ref, gw_ref, us_ref, ud_ref,
             oxp0_ref, oxp1_ref, oas_ref, oad_ref):
        h = _fin_block(l, acc36_ref[...], acc32_ref[...], st_ref[...],
                       as_ref[...], ad_ref[...], xp0_ref[...],
                       xp1_ref[...], b_ref[...])
        oxp0_ref[...], oxp1_ref[...], oas_ref[...], oad_ref[...] = (
            _prep_block(h, gw_ref[...], us_ref[...], ud_ref[...]))
    return body


def _finprep(l, acc36, acc32, stats, a_s, a_d, xp0, xp1, b, gW, us, ud):
    return pl.pallas_call(
        _make_finprep(l),
        grid=(N // NBLK,),
        in_specs=[
            pl.BlockSpec((2, NBLK, 36), lambda i: (0, i, 0)),
            pl.BlockSpec((2, NBLK, 32), lambda i: (0, i, 0)),
            pl.BlockSpec((2, NBLK, 16), lambda i: (0, i, 0)),
            pl.BlockSpec((NBLK, 16), lambda i: (i, 0)),
            pl.BlockSpec((NBLK, 16), lambda i: (i, 0)),
            pl.BlockSpec((NBLK, 32), lambda i: (i, 0)),
            pl.BlockSpec((NBLK, 32), lambda i: (i, 0)),
            pl.BlockSpec((1, HID), lambda i: (0, 0)),
            pl.BlockSpec((HID, HID), lambda i: (0, 0)),
            pl.BlockSpec((HID, H), lambda i: (0, 0)),
            pl.BlockSpec((HID, H), lambda i: (0, 0)),
        ],
        out_specs=[
            pl.BlockSpec((NBLK, 32), lambda i: (i, 0)),
            pl.BlockSpec((NBLK, 32), lambda i: (i, 0)),
            pl.BlockSpec((NBLK, 16), lambda i: (i, 0)),
            pl.BlockSpec((NBLK, 16), lambda i: (i, 0)),
        ],
        out_shape=[
            jax.ShapeDtypeStruct((N, 32), jnp.float32),
            jax.ShapeDtypeStruct((N, 32), jnp.float32),
            jax.ShapeDtypeStruct((N, 16), jnp.float32),
            jax.ShapeDtypeStruct((N, 16), jnp.float32),
        ],
    )(acc36, acc32, stats, a_s, a_d, xp0, xp1, b.reshape(1, HID), gW,
      us, ud)


def _make_fin2g(l):
    def body(acc36_ref, acc32_ref, st_ref, as_ref, ad_ref, xp0_ref,
             xp1_ref, b_ref, g_ref):
        h = _fin_block(l, acc36_ref[...], acc32_ref[...], st_ref[...],
                       as_ref[...], ad_ref[...], xp0_ref[...],
                       xp1_ref[...], b_ref[...])

        @pl.when(pl.program_id(0) == 0)
        def _():
            g_ref[...] = jnp.zeros_like(g_ref)

        g_ref[...] += jnp.sum(h, axis=0, keepdims=True)
    return body


def _fin2g(l, acc36, acc32, stats, a_s, a_d, xp0, xp1, b):
    return pl.pallas_call(
        _make_fin2g(l),
        grid=(N // NBLK,),
        in_specs=[
            pl.BlockSpec((2, NBLK, 36), lambda i: (0, i, 0)),
            pl.BlockSpec((2, NBLK, 32), lambda i: (0, i, 0)),
            pl.BlockSpec((2, NBLK, 16), lambda i: (0, i, 0)),
            pl.BlockSpec((NBLK, 16), lambda i: (i, 0)),
            pl.BlockSpec((NBLK, 16), lambda i: (i, 0)),
            pl.BlockSpec((NBLK, 32), lambda i: (i, 0)),
            pl.BlockSpec((NBLK, 32), lambda i: (i, 0)),
            pl.BlockSpec((1, HID), lambda i: (0, 0)),
        ],
        out_specs=pl.BlockSpec((1, HID), lambda i: (0, 0)),
        out_shape=jax.ShapeDtypeStruct((1, HID), jnp.float32),
    )(acc36, acc32, stats, a_s, a_d, xp0, xp1, b.reshape(1, HID))


_SIGCOLS = (5, 10, 11, 16, 17)


def _dec_body(g_ref, muW_ref, mub_ref, lvW_ref, lvb_ref, d0W_ref, d0b_ref,
              d0g_ref, d0e_ref, d1W_ref, d1b_ref, d1g_ref, d1e_ref,
              d2W_ref, d2b_ref, d2g_ref, d2e_ref, hW_ref, hb_ref,
              aW_ref, ab_ref, ho_ref, aux_ref, mu_ref, lv_ref):
    g = g_ref[...] / N
    mu = g @ muW_ref[...] + mub_ref[...]
    lv = g @ lvW_ref[...] + lvb_ref[...]
    hd = mu
    for dW, db, dg, de in ((d0W_ref, d0b_ref, d0g_ref, d0e_ref),
                           (d1W_ref, d1b_ref, d1g_ref, d1e_ref),
                           (d2W_ref, d2b_ref, d2g_ref, d2e_ref)):
        hd = hd @ dW[...] + db[...]
        mn = hd.mean(-1, keepdims=True)
        vr = ((hd - mn) ** 2).mean(-1, keepdims=True)
        hd = (hd - mn) / jnp.sqrt(vr + 1e-5) * dg[...] + de[...]
        hd = jax.nn.relu(hd)
    ho = hd @ hW_ref[...] + hb_ref[...]
    col = lax.broadcasted_iota(jnp.int32, ho.shape, 1)
    sigmask = jnp.zeros(ho.shape, jnp.bool_)
    for c in _SIGCOLS:
        sigmask = jnp.logical_or(sigmask, col == c)
    ho_ref[...] = jnp.where(sigmask, jax.nn.sigmoid(ho), ho)
    aux_ref[...] = mu @ aW_ref[...] + ab_ref[...]
    mu_ref[...] = mu
    lv_ref[...] = lv


def _decode(gsum, p, headsW, headsb):
    full = lambda s: pl.BlockSpec(s, lambda: tuple(0 for _ in s))
    args = (gsum, p["mu_W"], p["mu_b"].reshape(1, LAT), p["lv_W"],
            p["lv_b"].reshape(1, LAT),
            p["d0_W"], p["d0_b"].reshape(1, DH), p["d0_g"].reshape(1, DH),
            p["d0_be"].reshape(1, DH),
            p["d1_W"], p["d1_b"].reshape(1, DH), p["d1_g"].reshape(1, DH),
            p["d1_be"].reshape(1, DH),
            p["d2_W"], p["d2_b"].reshape(1, DH), p["d2_g"].reshape(1, DH),
            p["d2_be"].reshape(1, DH),
            headsW, headsb, p["aux_W"], p["aux_b"].reshape(1, 4))
    return pl.pallas_call(
        _dec_body,
        in_specs=[full(a.shape) for a in args],
        out_specs=[full((1, 18)), full((1, 4)), full((1, LAT)),
                   full((1, LAT))],
        out_shape=[
            jax.ShapeDtypeStruct((1, 18), jnp.float32),
            jax.ShapeDtypeStruct((1, 4), jnp.float32),
            jax.ShapeDtypeStruct((1, LAT), jnp.float32),
            jax.ShapeDtypeStruct((1, LAT), jnp.float32),
        ],
    )(*args)


def _sc_stats_build(NC, NS):
    NW = NC * NS
    MAXJ = -(-NCH // NW)

    @functools.partial(
        pl.kernel,
        out_type=jax.ShapeDtypeStruct((NC, N, 16), jnp.float32),
        mesh=_sc_mesh(),
        compiler_params=_SC_PARAMS,
        scratch_types=dict(
            idxd=pltpu.VMEM((1, CH), jnp.int32),
            rows=pltpu.VMEM((CH, 16), jnp.float32),
            statS=pltpu.VMEM_SHARED((N, 16), jnp.float32),
        ),
    )
    def k(ad_h, dst2_h, z16_h, out_h, *, idxd, rows, statS):
        c = lax.axis_index("c")
        s = lax.axis_index("s")
        wid = s * NC + c

        @pl.when(s == 0)
        def _():
            pltpu.sync_copy(z16_h, statS)

        plsc.subcore_barrier()

        def chunk(j, _):
            cid = j * NW + wid

            @pl.when(cid < NCH)
            def _():
                pltpu.sync_copy(dst2_h.at[pl.ds(cid, 1)], idxd)
                pltpu.sync_copy(ad_h.at[pl.ds(cid * CH, CH)], rows)
                pltpu.sync_copy(rows, statS.at[idxd.at[0]], add=True)

            return ()

        lax.fori_loop(0, MAXJ, chunk, ())
        plsc.subcore_barrier()

        @pl.when(s == 0)
        def _():
            pltpu.sync_copy(statS, out_h.at[c])

    return k


def _sc_pass0_build(NC, NS):
    NW = NC * NS
    MAXJ = -(-NCH // NW)

    @functools.partial(
        pl.kernel,
        out_type=[
            jax.ShapeDtypeStruct((E, 16), jnp.float32),
            jax.ShapeDtypeStruct((NC, N, 36), jnp.float32),
        ],
        mesh=_sc_mesh(),
        compiler_params=_SC_PARAMS,
        scratch_types=dict(
            idxs=pltpu.VMEM((1, CH), jnp.int32),
            idxd=pltpu.VMEM((1, CH), jnp.int32),
            aehg=pltpu.VMEM((CH * 8 + 16,), jnp.float32),
            asrcg=pltpu.VMEM((CH, 16), jnp.float32),
            adstg=pltpu.VMEM((CH, 16), jnp.float32),
            xpg=pltpu.VMEM((CH, 32), jnp.float32),
            srow=pltpu.VMEM((CH, 36), jnp.float32),
            wrow=pltpu.VMEM((CH, 16), jnp.float32),
            accS=pltpu.VMEM_SHARED((N, 36), jnp.float32),
            sem=pltpu.SemaphoreType.DMA,
            sem2=pltpu.SemaphoreType.DMA,
            sem3=pltpu.SemaphoreType.DMA,
        ),
    )
    def k(src2_h, dst2_h, aeh8_h, asrc_h, adst_h, xp0_h, z36_h,
          w_out, acc_out, *, idxs, idxd, aehg, asrcg, adstg, xpg, srow,
          wrow, accS, sem, sem2, sem3):
        c = lax.axis_index("c")
        s = lax.axis_index("s")
        wid = s * NC + c

        @pl.when(s == 0)
        def _():
            pltpu.sync_copy(z36_h, accS)

        plsc.subcore_barrier()

        def chunk(j, _):
            cid = j * NW + wid

            @pl.when(cid < NCH)
            def _():
                pltpu.sync_copy(src2_h.at[pl.ds(cid, 1)], idxs)
                pltpu.sync_copy(dst2_h.at[pl.ds(cid, 1)], idxd)
                pltpu.sync_copy(aeh8_h.at[pl.ds(cid * CH * 8, CH * 8)],
                                aehg.at[pl.ds(0, CH * 8)])
                cp1 = pltpu.async_copy(asrc_h.at[idxs.at[0]], asrcg, sem)
                cp2 = pltpu.async_copy(adst_h.at[idxd.at[0]], adstg, sem2)
                cp3 = pltpu.async_copy(xp0_h.at[idxs.at[0]], xpg, sem3)
                cp1.wait()
                cp2.wait()
                cp3.wait()

                def estep(e, _):
                    al = (asrcg[e, :] + adstg[e, :]
                          + aehg[pl.ds(e * 8, 16)])
                    al = jnp.where(al >= 0, al, 0.2 * al)
                    w16 = jnp.exp(al)
                    wrow[e, :] = w16
                    srow[e, pl.ds(0, 16)] = w16
                    x0 = xpg[e, pl.ds(0, 16)]
                    x1 = xpg[e, pl.ds(16, 16)]
                    srow[e, pl.ds(4, 16)] = x0 * w16[0]
                    srow[e, pl.ds(20, 16)] = x1 * w16[1]
                    return ()

                lax.fori_loop(0, CH, estep, (), unroll=4)

                pltpu.sync_copy(wrow, w_out.at[pl.ds(cid * CH, CH)])
                pltpu.sync_copy(srow, accS.at[idxd.at[0]], add=True)

            return ()

        lax.fori_loop(0, MAXJ, chunk, ())
        plsc.subcore_barrier()

        @pl.when(s == 0)
        def _():
            pltpu.sync_copy(accS, acc_out.at[c])

    return k


def _sc_pass1_build(NC, NS):
    NW = NC * NS
    MAXJP = (-(-NCH // NW) + 1) // 2

    @functools.partial(
        pl.kernel,
        out_type=jax.ShapeDtypeStruct((NC, N, 32), jnp.float32),
        mesh=_sc_mesh(),
        compiler_params=_SC_PARAMS,
        scratch_types=dict(
            idxsA=pltpu.VMEM((1, CH), jnp.int32),
            idxdA=pltpu.VMEM((1, CH), jnp.int32),
            wgA=pltpu.VMEM((CH, 16), jnp.float32),
            xpgA=pltpu.VMEM((CH, 32), jnp.float32),
            srowA=pltpu.VMEM((CH, 32), jnp.float32),
            idxsB=pltpu.VMEM((1, CH), jnp.int32),
            idxdB=pltpu.VMEM((1, CH), jnp.int32),
            wgB=pltpu.VMEM((CH, 16), jnp.float32),
            xpgB=pltpu.VMEM((CH, 32), jnp.float32),
            srowB=pltpu.VMEM((CH, 32), jnp.float32),
            accS=pltpu.VMEM_SHARED((N, 32), jnp.float32),
            gsemA=pltpu.SemaphoreType.DMA,
            gsemB=pltpu.SemaphoreType.DMA,
            ssemA=pltpu.SemaphoreType.DMA,
            ssemB=pltpu.SemaphoreType.DMA,
        ),
    )
    def k(src2_h, dst2_h, w_h, xp1_h, z32_h, acc_out, *, idxsA, idxdA,
          wgA, xpgA, srowA, idxsB, idxdB, wgB, xpgB, srowB, accS,
          gsemA, gsemB, ssemA, ssemB):
        c = lax.axis_index("c")
        s = lax.axis_index("s")
        wid = s * NC + c

        @pl.when(s == 0)
        def _():
            pltpu.sync_copy(z32_h, accS)

        plsc.subcore_barrier()

        def load_and_gather(cid, idxs, idxd, wg, xpg, gsem):
            pltpu.sync_copy(src2_h.at[pl.ds(cid, 1)], idxs)
            pltpu.sync_copy(dst2_h.at[pl.ds(cid, 1)], idxd)
            pltpu.sync_copy(w_h.at[pl.ds(cid * CH, CH)], wg)
            return pltpu.async_copy(xp1_h.at[idxs.at[0]], xpg, gsem)

        def compute(wg, xpg, srow):
            def estep(e, _):
                v = wg[e, :]
                srow[e, pl.ds(0, 16)] = xpg[e, pl.ds(0, 16)] * v[2]
                srow[e, pl.ds(16, 16)] = xpg[e, pl.ds(16, 16)] * v[3]
                return ()

            lax.fori_loop(0, CH, estep, (), unroll=4)

        def pair(jj, _):
            cA = (2 * jj) * NW + wid
            cB = cA + NW
            gA = load_and_gather(cA, idxsA, idxdA, wgA, xpgA, gsemA)
            gA.wait()
            compute(wgA, xpgA, srowA)
            oA = pltpu.async_copy(srowA, accS.at[idxdA.at[0]], ssemA,
                                  add=True)

            @pl.when(cB < NCH)
            def _():
                gB = load_and_gather(cB, idxsB, idxdB, wgB, xpgB, gsemB)
                gB.wait()
                compute(wgB, xpgB, srowB)
                cpB = pltpu.async_copy(srowB, accS.at[idxdB.at[0]],
                                       ssemB, add=True)
                cpB.wait()

            oA.wait()
            return ()

        lax.fori_loop(0, MAXJP, pair, ())
        plsc.subcore_barrier()

        @pl.when(s == 0)
        def _():
            pltpu.sync_copy(accS, acc_out.at[c])

    return k


@functools.lru_cache(maxsize=None)
def _sc_kernels():
    NC, NS = _sc_info()
    return (_sc_stats_build(NC, NS), _sc_pass0_build(NC, NS),
            _sc_pass1_build(NC, NS))


def kernel(x, face_types, edge_index, edge_attr, params):
    p = params
    src2 = edge_index[0].astype(jnp.int32).reshape(NCH, CH)
    dst2 = edge_index[1].astype(jnp.int32).reshape(NCH, CH)

    def fold(W, a):
        return (W.reshape(HID, H, C) * a[0][None]).sum(-1)

    us = [fold(p["g%d_W" % l], p["g%d_as" % l]) for l in range(3)]
    ud = [fold(p["g%d_W" % l], p["g%d_ad" % l]) for l in range(3)]
    ve = [fold(p["g%d_We" % l], p["g%d_ae" % l]) for l in range(3)]
    V3 = jnp.concatenate(ve, axis=1)
    headsW = jnp.concatenate(
        [p[n + "_W"] for n in ("core", "fil", "file", "h10", "h11",
                               "h1e", "h20", "h21", "h2e")], axis=1)
    headsb = jnp.concatenate(
        [p[n + "_b"] for n in ("core", "fil", "file", "h10", "h11",
                               "h1e", "h20", "h21", "h2e")]).reshape(1, 18)

    k_stats, k_pass0, k_pass1 = _sc_kernels()

    z16 = jnp.zeros((N, 16), jnp.float32)
    z32 = jnp.zeros((N, 32), jnp.float32)
    z36 = jnp.zeros((N, 36), jnp.float32)

    aehdeg, aeh8 = _edge_dense(edge_attr, p["ee_W"], p["ee_b"], V3)
    stats = k_stats(aehdeg, dst2, z16)

    xp0, xp1, a_s, a_d = _encprep(x, face_types, p["emb"], p["ne_W"],
                                  p["ne_b"], p["g0_W"], us[0], ud[0])

    for l in range(3):
        aeh8f = aeh8[l].reshape(E * 8)
        wrows, acc36 = k_pass0(src2, dst2, aeh8f, a_s, a_d, xp0, z36)
        acc32 = k_pass1(src2, dst2, wrows, xp1, z32)
        b = p["g%d_b" % l]
        if l < 2:
            xp0, xp1, a_s, a_d = _finprep(
                l, acc36, acc32, stats, a_s, a_d, xp0, xp1, b,
                p["g%d_W" % (l + 1)], us[l + 1], ud[l + 1])
        else:
            gsum = _fin2g(l, acc36, acc32, stats, a_s, a_d, xp0, xp1, b)

    ho, aux, mu, lv = _decode(gsum, p, headsW, headsb)

    core = ho[:, 0:4]
    fr = ho[:, 4:5]
    fx = ho[:, 5:6]
    h1 = jnp.stack([ho[:, 6:8], ho[:, 8:10]], axis=1)
    h1e = ho[:, 10:12]
    h2 = jnp.stack([ho[:, 12:14], ho[:, 14:16]], axis=1)
    h2e = ho[:, 16:18]
    return (core, fr, fx, h1, h1e, h2, h2e, aux, mu, lv)


# speedup vs baseline: 1.1695x; 1.0010x over previous
import functools

import jax
import jax.numpy as jnp
from jax import lax
from jax.experimental import pallas as pl
from jax.experimental.pallas import tpu as pltpu
from jax.experimental.pallas import tpu_sc as plsc

N = 50000
E = 800000
H = 4
C = 16
HID = 64
LAT = 32
DH = 256

NBLK = 1000
EBLK = 1000
CH = 128
NCH = E // CH


def _leaky(v):
    return jnp.where(v >= 0, v, 0.2 * v)


@functools.lru_cache(maxsize=None)
def _sc_info():
    info = plsc.get_sparse_core_info()
    return info.num_cores, info.num_subcores


def _sc_mesh():
    return plsc.VectorSubcoreMesh(core_axis_name="c", subcore_axis_name="s")


_SC_PARAMS = pltpu.CompilerParams(use_tc_tiling_on_sc=False)


def _edge_body(ea_ref, w_ref, b_ref, v3_ref, ad_ref, a8_ref):
    eh = jax.nn.relu(ea_ref[...] @ w_ref[...] + b_ref[...])
    a12 = eh @ v3_ref[...]
    one = jnp.ones((EBLK, 1), jnp.float32)
    zpad = jnp.zeros((EBLK, 3), jnp.float32)
    ad_ref[...] = jnp.concatenate([a12, one, zpad], axis=1)
    z4 = jnp.zeros((EBLK, 4), jnp.float32)
    a8_ref[...] = jnp.stack(
        [jnp.concatenate([a12[:, 4 * l:4 * l + 4], z4], axis=1)
         for l in range(3)], axis=0)


def _edge_dense(edge_attr, ee_W, ee_b, V3):
    return pl.pallas_call(
        _edge_body,
        grid=(E // EBLK,),
        in_specs=[
            pl.BlockSpec((EBLK, 2), lambda i: (i, 0)),
            pl.BlockSpec((2, HID), lambda i: (0, 0)),
            pl.BlockSpec((1, HID), lambda i: (0, 0)),
            pl.BlockSpec((HID, 12), lambda i: (0, 0)),
        ],
        out_specs=[
            pl.BlockSpec((EBLK, 16), lambda i: (i, 0)),
            pl.BlockSpec((3, EBLK, 8), lambda i: (0, i, 0)),
        ],
        out_shape=[
            jax.ShapeDtypeStruct((E, 16), jnp.float32),
            jax.ShapeDtypeStruct((3, E, 8), jnp.float32),
        ],
    )(edge_attr, ee_W, ee_b.reshape(1, HID), V3)


def _prep_block(h, W, us, ud):
    xp = h @ W
    z12 = jnp.zeros((h.shape[0], 12), jnp.float32)
    a_s = jnp.concatenate([h @ us, z12], axis=1)
    a_d = jnp.concatenate([h @ ud, z12], axis=1)
    return xp[:, :32], xp[:, 32:], a_s, a_d


def _encprep_body(x_ref, ft_ref, emb_ref, w_ref, b_ref, gw_ref, us_ref,
                  ud_ref, xp0_ref, xp1_ref, as_ref, ad_ref):
    ft = ft_ref[...]
    oh = (ft == lax.broadcasted_iota(jnp.int32, (ft.shape[0], 3), 1))
    fe = oh.astype(jnp.float32) @ emb_ref[...]
    hin = jnp.concatenate([x_ref[...], fe], axis=-1)
    h = jax.nn.relu(hin @ w_ref[...] + b_ref[...])
    xp0_ref[...], xp1_ref[...], as_ref[...], ad_ref[...] = _prep_block(
        h, gw_ref[...], us_ref[...], ud_ref[...])


def _encprep(x, face_types, emb, ne_W, ne_b, gW, us, ud):
    return pl.pallas_call(
        _encprep_body,
        grid=(N // NBLK,),
        in_specs=[
            pl.BlockSpec((NBLK, 9), lambda i: (i, 0)),
            pl.BlockSpec((NBLK, 1), lambda i: (i, 0)),
            pl.BlockSpec((3, 8), lambda i: (0, 0)),
            pl.BlockSpec((17, HID), lambda i: (0, 0)),
            pl.BlockSpec((1, HID), lambda i: (0, 0)),
            pl.BlockSpec((HID, HID), lambda i: (0, 0)),
            pl.BlockSpec((HID, H), lambda i: (0, 0)),
            pl.BlockSpec((HID, H), lambda i: (0, 0)),
        ],
        out_specs=[
            pl.BlockSpec((NBLK, 32), lambda i: (i, 0)),
            pl.BlockSpec((NBLK, 32), lambda i: (i, 0)),
            pl.BlockSpec((NBLK, 16), lambda i: (i, 0)),
            pl.BlockSpec((NBLK, 16), lambda i: (i, 0)),
        ],
        out_shape=[
            jax.ShapeDtypeStruct((N, 32), jnp.float32),
            jax.ShapeDtypeStruct((N, 32), jnp.float32),
            jax.ShapeDtypeStruct((N, 16), jnp.float32),
            jax.ShapeDtypeStruct((N, 16), jnp.float32),
        ],
    )(x, face_types.astype(jnp.int32).reshape(N, 1), emb, ne_W,
      ne_b.reshape(1, HID), gW, us, ud)


def _fin_block(l, acc36, acc32, st, a_s, a_d, xp0, xp1, b):
    a36 = acc36[0] + acc36[1]
    a32 = acc32[0] + acc32[1]
    stt = st[0] + st[1]
    deg = jnp.maximum(stt[:, 12:13], 1.0)
    aloop = stt[:, 4 * l:4 * l + 4] / deg
    wself = jnp.exp(_leaky(a_s[:, :4] + a_d[:, :4] + aloop))
    s = a36[:, 0:4] + wself
    acc = jnp.concatenate([a36[:, 4:36], a32], axis=1)
    B = acc.shape[0]
    xp = jnp.concatenate([xp0, xp1], axis=1).reshape(B, H, C)
    accf = acc.reshape(B, H, C) + xp * wself[:, :, None]
    out = accf / (s + 1e-16)[:, :, None]
    return jax.nn.relu(out.reshape(B, HID) + b)


def _make_finprep(l):
    def body(acc36_ref, acc32_ref, st_ref, as_ref, ad_ref, xp0_ref,
             xp1_ref, b_ref, gw_ref, us_ref, ud_ref,
             oxp0_ref, oxp1_ref, oas_ref, oad_ref):
        h = _fin_block(l, acc36_ref[...], acc32_ref[...], st_ref[...],
                       as_ref[...], ad_ref[...], xp0_ref[...],
                       xp1_ref[...], b_ref[...])
        oxp0_ref[...], oxp1_ref[...], oas_ref[...], oad_ref[...] = (
            _prep_block(h, gw_ref[...], us_ref[...], ud_ref[...]))
    return body


def _finprep(l, acc36, acc32, stats, a_s, a_d, xp0, xp1, b, gW, us, ud):
    return pl.pallas_call(
        _make_finprep(l),
        grid=(N // NBLK,),
        in_specs=[
            pl.BlockSpec((2, NBLK, 36), lambda i: (0, i, 0)),
            pl.BlockSpec((2, NBLK, 32), lambda i: (0, i, 0)),
            pl.BlockSpec((2, NBLK, 16), lambda i: (0, i, 0)),
            pl.BlockSpec((NBLK, 16), lambda i: (i, 0)),
            pl.BlockSpec((NBLK, 16), lambda i: (i, 0)),
            pl.BlockSpec((NBLK, 32), lambda i: (i, 0)),
            pl.BlockSpec((NBLK, 32), lambda i: (i, 0)),
            pl.BlockSpec((1, HID), lambda i: (0, 0)),
            pl.BlockSpec((HID, HID), lambda i: (0, 0)),
            pl.BlockSpec((HID, H), lambda i: (0, 0)),
            pl.BlockSpec((HID, H), lambda i: (0, 0)),
        ],
        out_specs=[
            pl.BlockSpec((NBLK, 32), lambda i: (i, 0)),
            pl.BlockSpec((NBLK, 32), lambda i: (i, 0)),
            pl.BlockSpec((NBLK, 16), lambda i: (i, 0)),
            pl.BlockSpec((NBLK, 16), lambda i: (i, 0)),
        ],
        out_shape=[
            jax.ShapeDtypeStruct((N, 32), jnp.float32),
            jax.ShapeDtypeStruct((N, 32), jnp.float32),
            jax.ShapeDtypeStruct((N, 16), jnp.float32),
            jax.ShapeDtypeStruct((N, 16), jnp.float32),
        ],
    )(acc36, acc32, stats, a_s, a_d, xp0, xp1, b.reshape(1, HID), gW,
      us, ud)


def _make_fin2g(l):
    def body(acc36_ref, acc32_ref, st_ref, as_ref, ad_ref, xp0_ref,
             xp1_ref, b_ref, g_ref):
        h = _fin_block(l, acc36_ref[...], acc32_ref[...], st_ref[...],
                       as_ref[...], ad_ref[...], xp0_ref[...],
                       xp1_ref[...], b_ref[...])

        @pl.when(pl.program_id(0) == 0)
        def _():
            g_ref[...] = jnp.zeros_like(g_ref)

        g_ref[...] += jnp.sum(h, axis=0, keepdims=True)
    return body


def _fin2g(l, acc36, acc32, stats, a_s, a_d, xp0, xp1, b):
    return pl.pallas_call(
        _make_fin2g(l),
        grid=(N // NBLK,),
        in_specs=[
            pl.BlockSpec((2, NBLK, 36), lambda i: (0, i, 0)),
            pl.BlockSpec((2, NBLK, 32), lambda i: (0, i, 0)),
            pl.BlockSpec((2, NBLK, 16), lambda i: (0, i, 0)),
            pl.BlockSpec((NBLK, 16), lambda i: (i, 0)),
            pl.BlockSpec((NBLK, 16), lambda i: (i, 0)),
            pl.BlockSpec((NBLK, 32), lambda i: (i, 0)),
            pl.BlockSpec((NBLK, 32), lambda i: (i, 0)),
            pl.BlockSpec((1, HID), lambda i: (0, 0)),
        ],
        out_specs=pl.BlockSpec((1, HID), lambda i: (0, 0)),
        out_shape=jax.ShapeDtypeStruct((1, HID), jnp.float32),
    )(acc36, acc32, stats, a_s, a_d, xp0, xp1, b.reshape(1, HID))


_SIGCOLS = (5, 10, 11, 16, 17)


def _dec_body(g_ref, muW_ref, mub_ref, lvW_ref, lvb_ref, d0W_ref, d0b_ref,
              d0g_ref, d0e_ref, d1W_ref, d1b_ref, d1g_ref, d1e_ref,
              d2W_ref, d2b_ref, d2g_ref, d2e_ref, hW_ref, hb_ref,
              aW_ref, ab_ref, ho_ref, aux_ref, mu_ref, lv_ref):
    g = g_ref[...] / N
    mu = g @ muW_ref[...] + mub_ref[...]
    lv = g @ lvW_ref[...] + lvb_ref[...]
    hd = mu
    for dW, db, dg, de in ((d0W_ref, d0b_ref, d0g_ref, d0e_ref),
                           (d1W_ref, d1b_ref, d1g_ref, d1e_ref),
                           (d2W_ref, d2b_ref, d2g_ref, d2e_ref)):
        hd = hd @ dW[...] + db[...]
        mn = hd.mean(-1, keepdims=True)
        vr = ((hd - mn) ** 2).mean(-1, keepdims=True)
        hd = (hd - mn) / jnp.sqrt(vr + 1e-5) * dg[...] + de[...]
        hd = jax.nn.relu(hd)
    ho = hd @ hW_ref[...] + hb_ref[...]
    col = lax.broadcasted_iota(jnp.int32, ho.shape, 1)
    sigmask = jnp.zeros(ho.shape, jnp.bool_)
    for c in _SIGCOLS:
        sigmask = jnp.logical_or(sigmask, col == c)
    ho_ref[...] = jnp.where(sigmask, jax.nn.sigmoid(ho), ho)
    aux_ref[...] = mu @ aW_ref[...] + ab_ref[...]
    mu_ref[...] = mu
    lv_ref[...] = lv


def _decode(gsum, p, headsW, headsb):
    full = lambda s: pl.BlockSpec(s, lambda: tuple(0 for _ in s))
    args = (gsum, p["mu_W"], p["mu_b"].reshape(1, LAT), p["lv_W"],
            p["lv_b"].reshape(1, LAT),
            p["d0_W"], p["d0_b"].reshape(1, DH), p["d0_g"].reshape(1, DH),
            p["d0_be"].reshape(1, DH),
            p["d1_W"], p["d1_b"].reshape(1, DH), p["d1_g"].reshape(1, DH),
            p["d1_be"].reshape(1, DH),
            p["d2_W"], p["d2_b"].reshape(1, DH), p["d2_g"].reshape(1, DH),
            p["d2_be"].reshape(1, DH),
            headsW, headsb, p["aux_W"], p["aux_b"].reshape(1, 4))
    return pl.pallas_call(
        _dec_body,
        in_specs=[full(a.shape) for a in args],
        out_specs=[full((1, 18)), full((1, 4)), full((1, LAT)),
                   full((1, LAT))],
        out_shape=[
            jax.ShapeDtypeStruct((1, 18), jnp.float32),
            jax.ShapeDtypeStruct((1, 4), jnp.float32),
            jax.ShapeDtypeStruct((1, LAT), jnp.float32),
            jax.ShapeDtypeStruct((1, LAT), jnp.float32),
        ],
    )(*args)


def _sc_stats_build(NC, NS):
    NW = NC * NS
    MAXJP = (-(-NCH // NW) + 1) // 2

    @functools.partial(
        pl.kernel,
        out_type=jax.ShapeDtypeStruct((NC, N, 16), jnp.float32),
        mesh=_sc_mesh(),
        compiler_params=_SC_PARAMS,
        scratch_types=dict(
            idxdA=pltpu.VMEM((1, CH), jnp.int32),
            rowsA=pltpu.VMEM((CH, 16), jnp.float32),
            idxdB=pltpu.VMEM((1, CH), jnp.int32),
            rowsB=pltpu.VMEM((CH, 16), jnp.float32),
            statS=pltpu.VMEM_SHARED((N, 16), jnp.float32),
            ssemA=pltpu.SemaphoreType.DMA,
            ssemB=pltpu.SemaphoreType.DMA,
        ),
    )
    def k(ad_h, dst2_h, z16_h, out_h, *, idxdA, rowsA, idxdB, rowsB,
          statS, ssemA, ssemB):
        c = lax.axis_index("c")
        s = lax.axis_index("s")
        wid = s * NC + c

        @pl.when(s == 0)
        def _():
            pltpu.sync_copy(z16_h, statS)

        plsc.subcore_barrier()

        def pair(jj, _):
            cA = (2 * jj) * NW + wid
            cB = cA + NW
            pltpu.sync_copy(dst2_h.at[pl.ds(cA, 1)], idxdA)
            pltpu.sync_copy(ad_h.at[pl.ds(cA * CH, CH)], rowsA)
            oA = pltpu.async_copy(rowsA, statS.at[idxdA.at[0]], ssemA,
                                  add=True)

            @pl.when(cB < NCH)
            def _():
                pltpu.sync_copy(dst2_h.at[pl.ds(cB, 1)], idxdB)
                pltpu.sync_copy(ad_h.at[pl.ds(cB * CH, CH)], rowsB)
                oB = pltpu.async_copy(rowsB, statS.at[idxdB.at[0]],
                                      ssemB, add=True)
                oB.wait()

            oA.wait()
            return ()

        lax.fori_loop(0, MAXJP, pair, ())
        plsc.subcore_barrier()

        @pl.when(s == 0)
        def _():
            pltpu.sync_copy(statS, out_h.at[c])

    return k


def _sc_pass0_build(NC, NS):
    NW = NC * NS
    MAXJ = -(-NCH // NW)

    @functools.partial(
        pl.kernel,
        out_type=[
            jax.ShapeDtypeStruct((E, 16), jnp.float32),
            jax.ShapeDtypeStruct((NC, N, 36), jnp.float32),
        ],
        mesh=_sc_mesh(),
        compiler_params=_SC_PARAMS,
        scratch_types=dict(
            idxs=pltpu.VMEM((1, CH), jnp.int32),
            idxd=pltpu.VMEM((1, CH), jnp.int32),
            aehg=pltpu.VMEM((CH * 8 + 16,), jnp.float32),
            asrcg=pltpu.VMEM((CH, 16), jnp.float32),
            adstg=pltpu.VMEM((CH, 16), jnp.float32),
            xpg=pltpu.VMEM((CH, 32), jnp.float32),
            srow=pltpu.VMEM((CH, 36), jnp.float32),
            wrow=pltpu.VMEM((CH, 16), jnp.float32),
            accS=pltpu.VMEM_SHARED((N, 36), jnp.float32),
            sem=pltpu.SemaphoreType.DMA,
            sem2=pltpu.SemaphoreType.DMA,
            sem3=pltpu.SemaphoreType.DMA,
        ),
    )
    def k(src2_h, dst2_h, aeh8_h, asrc_h, adst_h, xp0_h, z36_h,
          w_out, acc_out, *, idxs, idxd, aehg, asrcg, adstg, xpg, srow,
          wrow, accS, sem, sem2, sem3):
        c = lax.axis_index("c")
        s = lax.axis_index("s")
        wid = s * NC + c

        @pl.when(s == 0)
        def _():
            pltpu.sync_copy(z36_h, accS)

        plsc.subcore_barrier()

        def chunk(j, _):
            cid = j * NW + wid

            @pl.when(cid < NCH)
            def _():
                pltpu.sync_copy(src2_h.at[pl.ds(cid, 1)], idxs)
                pltpu.sync_copy(dst2_h.at[pl.ds(cid, 1)], idxd)
                pltpu.sync_copy(aeh8_h.at[pl.ds(cid * CH * 8, CH * 8)],
                                aehg.at[pl.ds(0, CH * 8)])
                cp1 = pltpu.async_copy(asrc_h.at[idxs.at[0]], asrcg, sem)
                cp2 = pltpu.async_copy(adst_h.at[idxd.at[0]], adstg, sem2)
                cp3 = pltpu.async_copy(xp0_h.at[idxs.at[0]], xpg, sem3)
                cp1.wait()
                cp2.wait()
                cp3.wait()

                def estep(e, _):
                    al = (asrcg[e, :] + adstg[e, :]
                          + aehg[pl.ds(e * 8, 16)])
                    al = jnp.where(al >= 0, al, 0.2 * al)
                    w16 = jnp.exp(al)
                    wrow[e, :] = w16
                    srow[e, pl.ds(0, 16)] = w16
                    x0 = xpg[e, pl.ds(0, 16)]
                    x1 = xpg[e, pl.ds(16, 16)]
                    srow[e, pl.ds(4, 16)] = x0 * w16[0]
                    srow[e, pl.ds(20, 16)] = x1 * w16[1]
                    return ()

                lax.fori_loop(0, CH, estep, (), unroll=4)

                pltpu.sync_copy(wrow, w_out.at[pl.ds(cid * CH, CH)])
                pltpu.sync_copy(srow, accS.at[idxd.at[0]], add=True)

            return ()

        lax.fori_loop(0, MAXJ, chunk, ())
        plsc.subcore_barrier()

        @pl.when(s == 0)
        def _():
            pltpu.sync_copy(accS, acc_out.at[c])

    return k


def _sc_pass1_build(NC, NS):
    NW = NC * NS
    MAXJP = (-(-NCH // NW) + 1) // 2

    @functools.partial(
        pl.kernel,
        out_type=jax.ShapeDtypeStruct((NC, N, 32), jnp.float32),
        mesh=_sc_mesh(),
        compiler_params=_SC_PARAMS,
        scratch_types=dict(
            idxsA=pltpu.VMEM((1, CH), jnp.int32),
            idxdA=pltpu.VMEM((1, CH), jnp.int32),
            wgA=pltpu.VMEM((CH, 16), jnp.float32),
            xpgA=pltpu.VMEM((CH, 32), jnp.float32),
            srowA=pltpu.VMEM((CH, 32), jnp.float32),
            idxsB=pltpu.VMEM((1, CH), jnp.int32),
            idxdB=pltpu.VMEM((1, CH), jnp.int32),
            wgB=pltpu.VMEM((CH, 16), jnp.float32),
            xpgB=pltpu.VMEM((CH, 32), jnp.float32),
            srowB=pltpu.VMEM((CH, 32), jnp.float32),
            accS=pltpu.VMEM_SHARED((N, 32), jnp.float32),
            gsemA=pltpu.SemaphoreType.DMA,
            gsemB=pltpu.SemaphoreType.DMA,
            ssemA=pltpu.SemaphoreType.DMA,
            ssemB=pltpu.SemaphoreType.DMA,
        ),
    )
    def k(src2_h, dst2_h, w_h, xp1_h, z32_h, acc_out, *, idxsA, idxdA,
          wgA, xpgA, srowA, idxsB, idxdB, wgB, xpgB, srowB, accS,
          gsemA, gsemB, ssemA, ssemB):
        c = lax.axis_index("c")
        s = lax.axis_index("s")
        wid = s * NC + c

        @pl.when(s == 0)
        def _():
            pltpu.sync_copy(z32_h, accS)

        plsc.subcore_barrier()

        def load_and_gather(cid, idxs, idxd, wg, xpg, gsem):
            pltpu.sync_copy(src2_h.at[pl.ds(cid, 1)], idxs)
            pltpu.sync_copy(dst2_h.at[pl.ds(cid, 1)], idxd)
            pltpu.sync_copy(w_h.at[pl.ds(cid * CH, CH)], wg)
            return pltpu.async_copy(xp1_h.at[idxs.at[0]], xpg, gsem)

        def compute(wg, xpg, srow):
            def estep(e, _):
                v = wg[e, :]
                srow[e, pl.ds(0, 16)] = xpg[e, pl.ds(0, 16)] * v[2]
                srow[e, pl.ds(16, 16)] = xpg[e, pl.ds(16, 16)] * v[3]
                return ()

            lax.fori_loop(0, CH, estep, (), unroll=4)

        def pair(jj, _):
            cA = (2 * jj) * NW + wid
            cB = cA + NW
            gA = load_and_gather(cA, idxsA, idxdA, wgA, xpgA, gsemA)
            gA.wait()
            compute(wgA, xpgA, srowA)
            oA = pltpu.async_copy(srowA, accS.at[idxdA.at[0]], ssemA,
                                  add=True)

            @pl.when(cB < NCH)
            def _():
                gB = load_and_gather(cB, idxsB, idxdB, wgB, xpgB, gsemB)
                gB.wait()
                compute(wgB, xpgB, srowB)
                cpB = pltpu.async_copy(srowB, accS.at[idxdB.at[0]],
                                       ssemB, add=True)
                cpB.wait()

            oA.wait()
            return ()

        lax.fori_loop(0, MAXJP, pair, ())
        plsc.subcore_barrier()

        @pl.when(s == 0)
        def _():
            pltpu.sync_copy(accS, acc_out.at[c])

    return k


@functools.lru_cache(maxsize=None)
def _sc_kernels():
    NC, NS = _sc_info()
    return (_sc_stats_build(NC, NS), _sc_pass0_build(NC, NS),
            _sc_pass1_build(NC, NS))


def kernel(x, face_types, edge_index, edge_attr, params):
    p = params
    src2 = edge_index[0].astype(jnp.int32).reshape(NCH, CH)
    dst2 = edge_index[1].astype(jnp.int32).reshape(NCH, CH)

    def fold(W, a):
        return (W.reshape(HID, H, C) * a[0][None]).sum(-1)

    us = [fold(p["g%d_W" % l], p["g%d_as" % l]) for l in range(3)]
    ud = [fold(p["g%d_W" % l], p["g%d_ad" % l]) for l in range(3)]
    ve = [fold(p["g%d_We" % l], p["g%d_ae" % l]) for l in range(3)]
    V3 = jnp.concatenate(ve, axis=1)
    headsW = jnp.concatenate(
        [p[n + "_W"] for n in ("core", "fil", "file", "h10", "h11",
                               "h1e", "h20", "h21", "h2e")], axis=1)
    headsb = jnp.concatenate(
        [p[n + "_b"] for n in ("core", "fil", "file", "h10", "h11",
                               "h1e", "h20", "h21", "h2e")]).reshape(1, 18)

    k_stats, k_pass0, k_pass1 = _sc_kernels()

    z16 = jnp.zeros((N, 16), jnp.float32)
    z32 = jnp.zeros((N, 32), jnp.float32)
    z36 = jnp.zeros((N, 36), jnp.float32)

    aehdeg, aeh8 = _edge_dense(edge_attr, p["ee_W"], p["ee_b"], V3)
    stats = k_stats(aehdeg, dst2, z16)

    xp0, xp1, a_s, a_d = _encprep(x, face_types, p["emb"], p["ne_W"],
                                  p["ne_b"], p["g0_W"], us[0], ud[0])

    for l in range(3):
        aeh8f = aeh8[l].reshape(E * 8)
        wrows, acc36 = k_pass0(src2, dst2, aeh8f, a_s, a_d, xp0, z36)
        acc32 = k_pass1(src2, dst2, wrows, xp1, z32)
        b = p["g%d_b" % l]
        if l < 2:
            xp0, xp1, a_s, a_d = _finprep(
                l, acc36, acc32, stats, a_s, a_d, xp0, xp1, b,
                p["g%d_W" % (l + 1)], us[l + 1], ud[l + 1])
        else:
            gsum = _fin2g(l, acc36, acc32, stats, a_s, a_d, xp0, xp1, b)

    ho, aux, mu, lv = _decode(gsum, p, headsW, headsb)

    core = ho[:, 0:4]
    fr = ho[:, 4:5]
    fx = ho[:, 5:6]
    h1 = jnp.stack([ho[:, 6:8], ho[:, 8:10]], axis=1)
    h1e = ho[:, 10:12]
    h2 = jnp.stack([ho[:, 12:14], ho[:, 14:16]], axis=1)
    h2e = ho[:, 16:18]
    return (core, fr, fx, h1, h1e, h2, h2e, aux, mu, lv)
